# baseline (speedup 1.0000x reference)
import functools

import jax
import jax.numpy as jnp
from jax import lax
from jax.experimental import pallas as pl
from jax.experimental.pallas import tpu as pltpu
from jax.experimental.pallas import tpu_sc as plsc


def _table_body(emb_ref, w_ref, b_ref, tab_ref):
    tab_ref[...] = lax.dot_general(
        emb_ref[...], w_ref[...],
        dimension_numbers=(((1,), (1,)), ((), ())),
        preferred_element_type=jnp.float32,
    ) + b_ref[...]


def _make_table(emb, w, b):
    V = w.shape[0]
    return pl.pallas_call(
        _table_body,
        out_shape=jax.ShapeDtypeStruct((emb.shape[0], V), jnp.float32),
    )(emb, w, b.reshape(1, V))


def _sc_gather(table, ids2d):
    B, L = ids2d.shape
    V = table.shape[1]
    info = plsc.get_sparse_core_info()
    nc, ns = info.num_cores, info.num_subcores
    nw = nc * ns
    rpw = B // nw
    nch = rpw
    mesh = plsc.VectorSubcoreMesh(core_axis_name="c", subcore_axis_name="s")

    @functools.partial(
        pl.kernel, mesh=mesh,
        compiler_params=pltpu.CompilerParams(use_tc_tiling_on_sc=False),
        out_type=jax.ShapeDtypeStruct((B, L, V), jnp.float32),
        scratch_types=[
            pltpu.VMEM((rpw, L), jnp.int32),
            pltpu.VMEM((L, V), jnp.float32),
            pltpu.VMEM((L, V), jnp.float32),
            pltpu.SemaphoreType.DMA,
            pltpu.SemaphoreType.DMA,
            pltpu.SemaphoreType.DMA,
            pltpu.SemaphoreType.DMA,
        ],
    )
    def k(tab_hbm, ids_hbm, out_hbm, idx_v, buf_a, buf_b, sga, sgb, soa, sob):
        wid = lax.axis_index("s") * nc + lax.axis_index("c")
        base = wid * rpw
        pltpu.sync_copy(ids_hbm.at[pl.ds(base, rpw)], idx_v)

        def g_start(c, buf, sem):
            pltpu.make_async_copy(tab_hbm.at[idx_v.at[c]], buf, sem).start()

        def g_wait(c, buf, sem):
            pltpu.make_async_copy(tab_hbm.at[idx_v.at[c]], buf, sem).wait()

        def o_start(c, buf, sem):
            pltpu.make_async_copy(buf, out_hbm.at[base + c], sem).start()

        def o_wait(c, buf, sem):
            pltpu.make_async_copy(buf, out_hbm.at[base + c], sem).wait()

        g_start(0, buf_a, sga)

        def body(i, carry):
            c = 2 * i
            g_wait(c, buf_a, sga)

            @pl.when(i > 0)
            def _():
                o_wait(c - 1, buf_b, sob)

            g_start(c + 1, buf_b, sgb)
            o_start(c, buf_a, soa)
            g_wait(c + 1, buf_b, sgb)
            o_wait(c, buf_a, soa)

            @pl.when(i < nch // 2 - 1)
            def _():
                g_start(c + 2, buf_a, sga)

            o_start(c + 1, buf_b, sob)
            return carry

        lax.fori_loop(0, nch // 2, body, 0)
        o_wait(nch - 1, buf_b, sob)

    return k(table, ids2d)


def kernel(input_ids, emb_table, lm_head_w, lm_head_b):
    table = _make_table(emb_table, lm_head_w, lm_head_b)
    return _sc_gather(table, input_ids)


# speedup vs baseline: 1.1136x; 1.1136x over previous
import functools

import jax
import jax.numpy as jnp
from jax import lax
from jax.experimental import pallas as pl
from jax.experimental.pallas import tpu as pltpu
from jax.experimental.pallas import tpu_sc as plsc


def _table_body(emb_ref, w_ref, b_ref, tab_ref):
    tab_ref[...] = lax.dot_general(
        emb_ref[...], w_ref[...],
        dimension_numbers=(((1,), (1,)), ((), ())),
        preferred_element_type=jnp.float32,
    ) + b_ref[...]


def _make_table(emb, w, b):
    V = w.shape[0]
    return pl.pallas_call(
        _table_body,
        out_shape=jax.ShapeDtypeStruct((emb.shape[0], V), jnp.float32),
    )(emb, w, b.reshape(1, V))


def _sc_gather(table, ids2d):
    B, L = ids2d.shape
    E, V = table.shape
    info = plsc.get_sparse_core_info()
    nc, ns = info.num_cores, info.num_subcores
    nw = nc * ns
    L2 = L // 2
    R = B * 2
    rpw = R // nw
    nch = rpw
    ids_r = ids2d.reshape(R, L2)
    tpw = (E + ns - 1) // ns
    mesh = plsc.VectorSubcoreMesh(core_axis_name="c", subcore_axis_name="s")

    @functools.partial(
        pl.kernel, mesh=mesh,
        compiler_params=pltpu.CompilerParams(use_tc_tiling_on_sc=False),
        out_type=jax.ShapeDtypeStruct((R, L2, V), jnp.float32),
        scratch_types=[
            pltpu.VMEM((rpw, L2), jnp.int32),
            pltpu.VMEM((L2, V), jnp.float32),
            pltpu.VMEM((L2, V), jnp.float32),
            pltpu.VMEM_SHARED((E, V), jnp.float32),
            pltpu.SemaphoreType.DMA,
            pltpu.SemaphoreType.DMA,
            pltpu.SemaphoreType.DMA,
            pltpu.SemaphoreType.DMA,
        ],
    )
    def k(tab_hbm, ids_hbm, out_hbm, idx_v, buf_a, buf_b, tab_sp, sga, sgb,
          soa, sob):
        cid = lax.axis_index("c")
        sid = lax.axis_index("s")
        wid = sid * nc + cid
        base = wid * rpw
        pltpu.sync_copy(ids_hbm.at[pl.ds(base, rpw)], idx_v)

        t0 = sid * tpw

        @pl.when(t0 + tpw <= E)
        def _():
            pltpu.sync_copy(tab_hbm.at[pl.ds(t0, tpw)],
                            tab_sp.at[pl.ds(t0, tpw)])

        @pl.when(t0 + tpw > E)
        def _():
            last = E - (ns - 1) * tpw
            pltpu.sync_copy(tab_hbm.at[pl.ds(t0, last)],
                            tab_sp.at[pl.ds(t0, last)])

        plsc.subcore_barrier()

        def g_start(c, buf, sem):
            pltpu.make_async_copy(tab_sp.at[idx_v.at[c]], buf, sem).start()

        def g_wait(c, buf, sem):
            pltpu.make_async_copy(tab_sp.at[idx_v.at[c]], buf, sem).wait()

        def o_start(c, buf, sem):
            pltpu.make_async_copy(buf, out_hbm.at[base + c], sem).start()

        def o_wait(c, buf, sem):
            pltpu.make_async_copy(buf, out_hbm.at[base + c], sem).wait()

        g_start(0, buf_a, sga)

        def body(i, carry):
            c = 2 * i
            g_wait(c, buf_a, sga)

            @pl.when(i > 0)
            def _():
                o_wait(c - 1, buf_b, sob)

            g_start(c + 1, buf_b, sgb)
            o_start(c, buf_a, soa)
            g_wait(c + 1, buf_b, sgb)
            o_wait(c, buf_a, soa)

            @pl.when(i < nch // 2 - 1)
            def _():
                g_start(c + 2, buf_a, sga)

            o_start(c + 1, buf_b, sob)
            return carry

        lax.fori_loop(0, nch // 2, body, 0)
        o_wait(nch - 1, buf_b, sob)

    return k(table, ids_r).reshape(B, L, V)


def kernel(input_ids, emb_table, lm_head_w, lm_head_b):
    table = _make_table(emb_table, lm_head_w, lm_head_b)
    return _sc_gather(table, input_ids)


# speedup vs baseline: 1.1166x; 1.0027x over previous
import functools

import jax
import jax.numpy as jnp
from jax import lax
from jax.experimental import pallas as pl
from jax.experimental.pallas import tpu as pltpu
from jax.experimental.pallas import tpu_sc as plsc


def _table_body(emb_ref, w_ref, b_ref, tab_ref):
    tab_ref[...] = lax.dot_general(
        emb_ref[...], w_ref[...],
        dimension_numbers=(((1,), (1,)), ((), ())),
        preferred_element_type=jnp.float32,
    ) + b_ref[...]


def _make_table(emb, w, b):
    V = w.shape[0]
    return pl.pallas_call(
        _table_body,
        out_shape=jax.ShapeDtypeStruct((emb.shape[0], V), jnp.float32),
    )(emb, w, b.reshape(1, V))


def _sc_gather(table, ids2d):
    B, L = ids2d.shape
    E, V = table.shape
    info = plsc.get_sparse_core_info()
    nc, ns = info.num_cores, info.num_subcores
    nw = nc * ns
    CH = 16
    NB = 4
    R = (B * L) // CH
    rpw = R // nw
    ids_r = ids2d.reshape(R, CH)
    tpw = (E + ns - 1) // ns
    mesh = plsc.VectorSubcoreMesh(core_axis_name="c", subcore_axis_name="s")

    @functools.partial(
        pl.kernel, mesh=mesh,
        compiler_params=pltpu.CompilerParams(use_tc_tiling_on_sc=False),
        out_type=jax.ShapeDtypeStruct((R, CH, V), jnp.float32),
        scratch_types=(
            [pltpu.VMEM((rpw, CH), jnp.int32)]
            + [pltpu.VMEM((CH, V), jnp.float32) for _ in range(NB)]
            + [pltpu.VMEM_SHARED((E, V), jnp.float32)]
            + [pltpu.SemaphoreType.DMA for _ in range(2 * NB)]
        ),
    )
    def k(tab_hbm, ids_hbm, out_hbm, idx_v, b0, b1, b2, b3, tab_sp,
          g0, g1, g2, g3, o0, o1, o2, o3):
        bufs = [b0, b1, b2, b3]
        sgs = [g0, g1, g2, g3]
        sos = [o0, o1, o2, o3]
        cid = lax.axis_index("c")
        sid = lax.axis_index("s")
        wid = sid * nc + cid
        base = wid * rpw
        pltpu.sync_copy(ids_hbm.at[pl.ds(base, rpw)], idx_v)

        t0 = sid * tpw

        @pl.when(t0 + tpw <= E)
        def _():
            pltpu.sync_copy(tab_hbm.at[pl.ds(t0, tpw)],
                            tab_sp.at[pl.ds(t0, tpw)])

        @pl.when(t0 + tpw > E)
        def _():
            last = E - (ns - 1) * tpw
            pltpu.sync_copy(tab_hbm.at[pl.ds(t0, last)],
                            tab_sp.at[pl.ds(t0, last)])

        plsc.subcore_barrier()

        def g_copy(c, b):
            return pltpu.make_async_copy(
                tab_sp.at[idx_v.at[c]], bufs[b], sgs[b])

        def o_copy(c, b):
            return pltpu.make_async_copy(
                bufs[b], out_hbm.at[base + c], sos[b])

        for b in range(NB):
            g_copy(b, b).start()

        nloop = rpw // NB

        def body(i, carry):
            for b in range(NB):
                c = i * NB + b
                g_copy(c, b).wait()

                @pl.when(i > 0)
                def _():
                    o_copy(c - NB, b).wait()

                o_copy(c, b).start()

                @pl.when(i < nloop - 1)
                def _():
                    g_copy(c + NB, b).start()

            return carry

        lax.fori_loop(0, nloop, body, 0)
        for b in range(NB):
            o_copy(rpw - NB + b, b).wait()

    return k(table, ids_r).reshape(B, L, V)


def kernel(input_ids, emb_table, lm_head_w, lm_head_b):
    table = _make_table(emb_table, lm_head_w, lm_head_b)
    return _sc_gather(table, input_ids)
